# Initial kernel scaffold; baseline (speedup 1.0000x reference)
import jax
import jax.numpy as jnp
from jax.experimental import pallas as pl


def kernel(x, edge_index, W_in, b_in, W1, b1, gn1_w, gn1_b, gn1_ms, W2, b2, gn2_w, gn2_b, gn2_ms):
    raise NotImplementedError("write your pallas kernel here")


# speedup vs baseline: 18.5399x; 18.5399x over previous
import functools

import jax
import jax.numpy as jnp
from jax import lax
from jax.experimental import pallas as pl
from jax.experimental.pallas import tpu as pltpu
from jax.experimental.pallas import tpu_sc as plsc

N = 10000
E = 320000
D_IN = 128
H = 64

NC = 2
NS = 16
NW = NC * NS

CHUNK = 128
NCHUNK = E // CHUNK
ROWS_PER_TILE = N // NS
DEG_W = 16

_mesh = plsc.VectorSubcoreMesh(core_axis_name="c", subcore_axis_name="s")


def _worker_id():
    return lax.axis_index("s") * NC + lax.axis_index("c")


def _zero_fill(buf, nrows, ncols):
    zeros = jnp.zeros((16,), jnp.float32)

    def body(r, _):
        for c in range(ncols // 16):
            buf[r, pl.ds(c * 16, 16)] = zeros
        return 0

    lax.fori_loop(0, nrows, body, 0)


@functools.partial(
    pl.kernel,
    mesh=_mesh,
    compiler_params=pltpu.CompilerParams(use_tc_tiling_on_sc=False),
    out_type=jax.ShapeDtypeStruct((NC, N, DEG_W), jnp.float32),
    scratch_types=[
        pltpu.VMEM((CHUNK,), jnp.int32),
        pltpu.VMEM((CHUNK, DEG_W), jnp.float32),
        pltpu.VMEM((ROWS_PER_TILE, DEG_W), jnp.float32),
        pltpu.VMEM_SHARED((N, DEG_W), jnp.float32),
    ],
)
def _sc_degree(col_hbm, out_hbm, col_v, ones_v, stage_v, acc_sh):
    cid = lax.axis_index("c")
    sid = lax.axis_index("s")
    wid = _worker_id()

    ones = jnp.ones((16,), jnp.float32)

    def fill_ones(r, _):
        ones_v[r, pl.ds(0, 16)] = ones
        return 0

    lax.fori_loop(0, CHUNK, fill_ones, 0)

    _zero_fill(stage_v, ROWS_PER_TILE, DEG_W)
    row0 = sid * ROWS_PER_TILE
    pltpu.sync_copy(stage_v, acc_sh.at[pl.ds(row0, ROWS_PER_TILE), :])
    plsc.subcore_barrier()

    def body(k, _):
        chunk = wid + NW * k

        @pl.when(chunk < NCHUNK)
        def _():
            base = pl.multiple_of(chunk * CHUNK, CHUNK)
            pltpu.sync_copy(col_hbm.at[pl.ds(base, CHUNK)], col_v)
            pltpu.sync_copy(ones_v, acc_sh.at[col_v], add=True)

        return 0

    lax.fori_loop(0, (NCHUNK + NW - 1) // NW, body, 0)
    plsc.subcore_barrier()

    pltpu.sync_copy(acc_sh.at[pl.ds(row0, ROWS_PER_TILE), :], stage_v)
    pltpu.sync_copy(stage_v, out_hbm.at[cid, pl.ds(row0, ROWS_PER_TILE), :])


@functools.partial(
    pl.kernel,
    mesh=_mesh,
    compiler_params=pltpu.CompilerParams(use_tc_tiling_on_sc=False),
    out_type=jax.ShapeDtypeStruct((NC, N, H), jnp.float32),
    scratch_types=[
        pltpu.VMEM((CHUNK,), jnp.int32),
        pltpu.VMEM((CHUNK,), jnp.int32),
        pltpu.VMEM((CHUNK, H), jnp.float32),
        pltpu.VMEM((ROWS_PER_TILE, H), jnp.float32),
        pltpu.VMEM_SHARED((N, H), jnp.float32),
        pltpu.SemaphoreType.DMA,
    ],
)
def _sc_agg(y_hbm, row_hbm, col_hbm, out_hbm,
            row_v, col_v, rows_v, stage_v, acc_sh, sem):
    cid = lax.axis_index("c")
    sid = lax.axis_index("s")
    wid = _worker_id()

    _zero_fill(stage_v, ROWS_PER_TILE, H)
    row0 = sid * ROWS_PER_TILE
    pltpu.sync_copy(stage_v, acc_sh.at[pl.ds(row0, ROWS_PER_TILE), :])
    plsc.subcore_barrier()

    def body(k, _):
        chunk = wid + NW * k

        @pl.when(chunk < NCHUNK)
        def _():
            base = pl.multiple_of(chunk * CHUNK, CHUNK)
            pltpu.sync_copy(row_hbm.at[pl.ds(base, CHUNK)], row_v)
            pltpu.sync_copy(col_hbm.at[pl.ds(base, CHUNK)], col_v)
            pltpu.async_copy(y_hbm.at[row_v], rows_v, sem).wait()
            pltpu.sync_copy(rows_v, acc_sh.at[col_v], add=True)

        return 0

    lax.fori_loop(0, (NCHUNK + NW - 1) // NW, body, 0)
    plsc.subcore_barrier()

    pltpu.sync_copy(acc_sh.at[pl.ds(row0, ROWS_PER_TILE), :], stage_v)
    pltpu.sync_copy(stage_v, out_hbm.at[cid, pl.ds(row0, ROWS_PER_TILE), :])


def _dis_from_parts(degp_ref):
    deg = degp_ref[0, :, 0:1] + degp_ref[1, :, 0:1] + 1.0
    return lax.rsqrt(deg)


def _tc_pre_body(x_ref, win_ref, bin_ref, w1_ref, degp_ref, y_ref):
    dis = _dis_from_parts(degp_ref)
    h0 = jnp.dot(x_ref[...], win_ref[...],
                 preferred_element_type=jnp.float32) + bin_ref[...]
    y_ref[...] = dis * jnp.dot(h0, w1_ref[...],
                               preferred_element_type=jnp.float32)


def _graph_norm(o, w, b, ms, eps=1e-5):
    mean = jnp.mean(o, axis=0, keepdims=True)
    out = o - ms * mean
    var = jnp.mean(out * out, axis=0, keepdims=True)
    return w * out / jnp.sqrt(var + eps) + b


def _tc_mid_body(ap_ref, y1_ref, degp_ref, b1_ref, w_ref, b_ref, ms_ref,
                 w2_ref, y2_ref):
    dis = _dis_from_parts(degp_ref)
    o = dis * (ap_ref[0] + ap_ref[1] + y1_ref[...]) + b1_ref[...]
    g = _graph_norm(o, w_ref[...], b_ref[...], ms_ref[...])
    h1 = jnp.where(g >= 0, g, 0.2 * g)
    y2_ref[...] = dis * jnp.dot(h1, w2_ref[...],
                                preferred_element_type=jnp.float32)


def _tc_post_body(aq_ref, y2_ref, degp_ref, b2_ref, w_ref, b_ref, ms_ref,
                  out_ref):
    dis = _dis_from_parts(degp_ref)
    o = dis * (aq_ref[0] + aq_ref[1] + y2_ref[...]) + b2_ref[...]
    out_ref[...] = _graph_norm(o, w_ref[...], b_ref[...], ms_ref[...])


def kernel(x, edge_index, W_in, b_in, W1, b1, gn1_w, gn1_b, gn1_ms,
           W2, b2, gn2_w, gn2_b, gn2_ms):
    row = edge_index[0].astype(jnp.int32)
    col = edge_index[1].astype(jnp.int32)

    degp = _sc_degree(col)

    y1 = pl.pallas_call(
        _tc_pre_body,
        out_shape=jax.ShapeDtypeStruct((N, H), jnp.float32),
    )(x, W_in, b_in.reshape(1, H), W1, degp)

    ap = _sc_agg(y1, row, col)

    y2 = pl.pallas_call(
        _tc_mid_body,
        out_shape=jax.ShapeDtypeStruct((N, H), jnp.float32),
    )(ap, y1, degp, b1.reshape(1, H), gn1_w.reshape(1, H),
      gn1_b.reshape(1, H), gn1_ms.reshape(1, H), W2)

    aq = _sc_agg(y2, row, col)

    out = pl.pallas_call(
        _tc_post_body,
        out_shape=jax.ShapeDtypeStruct((N, H), jnp.float32),
    )(aq, y2, degp, b2.reshape(1, H), gn2_w.reshape(1, H),
      gn2_b.reshape(1, H), gn2_ms.reshape(1, H))

    return out
